# baseline (speedup 1.0000x reference)
import jax
import jax.numpy as jnp
from jax.experimental import pallas as pl
from jax.experimental.pallas import tpu as pltpu


def _trim_kernel(x_ref, v_ref, m_ref, xo_ref, vo_ref, mo_ref, sem_x, sem_v):
    cx = pltpu.make_async_copy(x_ref, xo_ref, sem_x)
    cv = pltpu.make_async_copy(v_ref, vo_ref, sem_v)
    cx.start()
    cv.start()
    mo_ref[...] = m_ref[...] != 0.0
    cx.wait()
    cv.wait()


def kernel(x, v, mask):
    xo, vo, mo = pl.pallas_call(
        _trim_kernel,
        in_specs=[
            pl.BlockSpec(memory_space=pl.ANY),
            pl.BlockSpec(memory_space=pl.ANY),
            pl.BlockSpec(memory_space=pltpu.MemorySpace.VMEM),
        ],
        out_specs=[
            pl.BlockSpec(memory_space=pl.ANY),
            pl.BlockSpec(memory_space=pl.ANY),
            pl.BlockSpec(memory_space=pltpu.MemorySpace.VMEM),
        ],
        out_shape=[
            jax.ShapeDtypeStruct(x.shape, x.dtype),
            jax.ShapeDtypeStruct(v.shape, v.dtype),
            jax.ShapeDtypeStruct(mask.shape, jnp.bool_),
        ],
        scratch_shapes=[pltpu.SemaphoreType.DMA, pltpu.SemaphoreType.DMA],
    )(x, v, mask)
    return (xo, vo, mo)


# speedup vs baseline: 26.7706x; 26.7706x over previous
import jax
import jax.numpy as jnp
from jax.experimental import pallas as pl
from jax.experimental.pallas import tpu as pltpu


def _trim_kernel(x_ref, v_ref, m_ref, xo_ref, vo_ref, mo_ref):
    xo_ref[...] = x_ref[...]
    vo_ref[...] = v_ref[...]
    mo_ref[...] = m_ref[...] != 0.0


def kernel(x, v, mask):
    b, n, l = x.shape
    _, nv, _ = v.shape
    _, nm, _ = mask.shape
    xo, vo, mo = pl.pallas_call(
        _trim_kernel,
        grid=(b,),
        in_specs=[
            pl.BlockSpec((1, n, l), lambda i: (i, 0, 0)),
            pl.BlockSpec((1, nv, l), lambda i: (i, 0, 0)),
            pl.BlockSpec((1, nm, l), lambda i: (i, 0, 0)),
        ],
        out_specs=[
            pl.BlockSpec((1, n, l), lambda i: (i, 0, 0)),
            pl.BlockSpec((1, nv, l), lambda i: (i, 0, 0)),
            pl.BlockSpec((1, nm, l), lambda i: (i, 0, 0)),
        ],
        out_shape=[
            jax.ShapeDtypeStruct(x.shape, x.dtype),
            jax.ShapeDtypeStruct(v.shape, v.dtype),
            jax.ShapeDtypeStruct(mask.shape, jnp.bool_),
        ],
        compiler_params=pltpu.CompilerParams(
            dimension_semantics=(pltpu.GridDimensionSemantics.PARALLEL,),
        ),
    )(x, v, mask)
    return (xo, vo, mo)
